# baseline (speedup 1.0000x reference)
import functools

import numpy as np
import jax
import jax.numpy as jnp
from jax import lax
from jax.experimental import pallas as pl
from jax.experimental.pallas import tpu as pltpu
from jax.experimental.pallas import tpu_sc as plsc

OUT_H, OUT_W = 8, 32
NSAMP = OUT_H * OUT_W
SPATIAL_SCALE = 0.25
N_IMG, C, H, W = 2, 128, 160, 160
R = 1000
RPAD = 1024
NC, NS = 2, 16
WORKERS = NC * NS
RPW = RPAD // WORKERS
SUB = 32
NSUB = NSAMP // SUB


def _coords_body(rois_ref, idx_ref, w_ref):
    r = rois_ref[...]
    b = r[:, 0:1].astype(jnp.int32)

    def pcol(i):
        return r[:, 1 + i:2 + i] * SPATIAL_SCALE

    px = [pcol(2 * j) for j in range(8)]
    py = [pcol(2 * j + 1) for j in range(8)]

    rb = r.shape[0]
    pos = lax.broadcasted_iota(jnp.int32, (rb, NSAMP), 1)
    u = (pos % OUT_W).astype(jnp.float32) * (1.0 / OUT_W)
    v = (pos // OUT_W).astype(jnp.float32) * (1.0 / OUT_H)

    def bez(p0, p1, p2, p3, t):
        omt = 1.0 - t
        return (omt * omt * omt) * p0 + 3.0 * (omt * omt) * t * p1 \
            + 3.0 * omt * (t * t) * p2 + (t * t * t) * p3

    x0 = bez(px[0], px[1], px[2], px[3], u)
    y0 = bez(py[0], py[1], py[2], py[3], u)
    x1 = bez(px[4], px[5], px[6], px[7], u)
    y1 = bez(py[4], py[5], py[6], py[7], u)
    xc = x1 * v + x0 * (1.0 - v) - 0.5
    yc = y1 * v + y0 * (1.0 - v) - 0.5

    roi_w = jnp.maximum(jnp.abs(px[0] - px[3]), jnp.abs(px[4] - px[7]))
    roi_h = jnp.maximum(jnp.abs(py[0] - py[4]), jnp.abs(py[3] - py[7]))
    bin_h = roi_h * (1.0 / OUT_H)
    bin_w = roi_w * (1.0 / OUT_W)
    ys = yc - 0.5 * bin_h + 0.5 * bin_h
    xs = xc - 0.5 * bin_w + 0.5 * bin_w

    valid = (ys >= -1.0) & (ys <= float(H)) & (xs >= -1.0) & (xs <= float(W))
    y = jnp.maximum(ys, 0.0)
    x = jnp.maximum(xs, 0.0)
    yl = jnp.minimum(y.astype(jnp.int32), H - 1)
    xl = jnp.minimum(x.astype(jnp.int32), W - 1)
    yh = jnp.minimum(yl + 1, H - 1)
    xh = jnp.minimum(xl + 1, W - 1)
    y = jnp.where(yl >= H - 1, yl.astype(jnp.float32), y)
    x = jnp.where(xl >= W - 1, xl.astype(jnp.float32), x)
    ly = y - yl.astype(jnp.float32)
    lx = x - xl.astype(jnp.float32)
    hy = 1.0 - ly
    hx = 1.0 - lx
    vf = valid.astype(jnp.float32)

    w_ref[:, 0, :] = hy * hx * vf
    w_ref[:, 1, :] = hy * lx * vf
    w_ref[:, 2, :] = ly * hx * vf
    w_ref[:, 3, :] = ly * lx * vf
    base = b * (H * W)
    idx_ref[:, 0, :] = base + yl * W + xl
    idx_ref[:, 1, :] = base + yh * W + xl


_RB = 128


def _coords(rois_p):
    return pl.pallas_call(
        _coords_body,
        grid=(RPAD // _RB,),
        in_specs=[pl.BlockSpec((_RB, 17), lambda i: (i, 0))],
        out_specs=[
            pl.BlockSpec((_RB, 2, NSAMP), lambda i: (i, 0, 0)),
            pl.BlockSpec((_RB, 4, NSAMP), lambda i: (i, 0, 0)),
        ],
        out_shape=[
            jax.ShapeDtypeStruct((RPAD, 2, NSAMP), jnp.int32),
            jax.ShapeDtypeStruct((RPAD, 4, NSAMP), jnp.float32),
        ],
    )(rois_p)


@functools.cache
def _sc_gather_fn():
    return functools.partial(
        pl.kernel,
        mesh=plsc.VectorSubcoreMesh(core_axis_name="c", subcore_axis_name="s"),
        out_type=jax.ShapeDtypeStruct((RPAD, C, NSAMP), jnp.float32),
        compiler_params=pltpu.CompilerParams(needs_layout_passes=False),
        scratch_types=[
            pltpu.VMEM((2, NSUB, SUB), jnp.int32),
            pltpu.VMEM((4 * NSAMP,), jnp.int32),
            pltpu.VMEM((2, SUB, C), jnp.int32),
            pltpu.VMEM((2, SUB, C), jnp.int32),
            pltpu.VMEM((C, NSAMP + 1), jnp.float32),
            pltpu.VMEM((C, NSAMP + 1), jnp.float32),
            pltpu.SemaphoreType.DMA,
            pltpu.SemaphoreType.DMA,
            pltpu.SemaphoreType.DMA,
        ],
    )(_sc_gather_body)


def _sc_gather_body(table, idxs, ws, out, idx0, w0, stage0, stage1,
                    acc0, acc1, sem0, sem1, semo):
    wid = lax.axis_index("s") * NC + lax.axis_index("c")
    rbase = wid * RPW

    def issue(idxr, j, stage, sem):
        for k in range(2):
            pltpu.async_copy(table.at[idxr.at[k, j]], stage.at[k], sem)

    def drain(stage, sem):
        for k in range(2):
            pltpu.make_async_copy(table.at[idx0.at[k, 0]], stage.at[k],
                                  sem).wait()

    def compute(j, stage, accr, wref):
        wbase = j * SUB
        wvecs = [[wref[pl.ds(k * NSAMP + wbase + h * 16, 16)]
                  for h in range(2)] for k in range(4)]
        ch_vecs = [lax.iota(jnp.int32, 16) + 32 * g for g in range(C // 32)]
        for s in range(SUB):
            wgt = [plsc.bitcast(
                jnp.full((16,), wvecs[k][s // 16][s % 16], jnp.int32),
                jnp.bfloat16) for k in range(4)]
            col_vec = jnp.full((16,), wbase + s, jnp.int32)
            for g in range(C // 32):
                accp = None
                for row, half, wi in ((0, 0, 0), (0, 1, 1), (1, 0, 2),
                                      (1, 1, 3)):
                    chunk = plsc.bitcast(
                        stage[row, s, pl.ds(half * (C // 2) + 16 * g, 16)],
                        jnp.bfloat16)
                    term = chunk * wgt[wi]
                    accp = term if accp is None else accp + term
                lo, hi = plsc.unpack(accp, format=plsc.PackFormat.INTERLEAVED)
                plsc.store_scatter(accr, [ch_vecs[g], col_vec], lo)
                plsc.store_scatter(accr, [ch_vecs[g] + 16, col_vec], hi)

    def do_roi(il, idxr, wref, accr):
        pltpu.sync_copy(idxs.at[rbase + il], idxr)
        pltpu.sync_copy(ws.at[rbase + il], wref)
        issue(idxr, 0, stage0, sem0)

        @pl.when(il >= 2)
        def _():
            pltpu.make_async_copy(accr.at[:, pl.ds(0, NSAMP)],
                                  out.at[rbase], semo).wait()

        def pair_body(jj, c2):
            j0 = jj * 2
            issue(idxr, j0 + 1, stage1, sem1)
            drain(stage0, sem0)
            compute(j0, stage0, accr, wref)

            @pl.when(jj < NSUB // 2 - 1)
            def _():
                issue(idxr, j0 + 2, stage0, sem0)

            drain(stage1, sem1)
            compute(j0 + 1, stage1, accr, wref)
            return c2

        lax.fori_loop(0, NSUB // 2, pair_body, 0)
        pltpu.async_copy(accr.at[:, pl.ds(0, NSAMP)], out.at[rbase + il],
                         semo)

    def pair_rois(i2, carry):
        do_roi(i2 * 2, idx0, w0, acc0)
        do_roi(i2 * 2 + 1, idx0, w0, acc1)
        return carry

    lax.fori_loop(0, RPW // 2, pair_rois, 0)
    for accr in (acc0, acc1):
        pltpu.make_async_copy(accr.at[:, pl.ds(0, NSAMP)], out.at[rbase],
                              semo).wait()


def kernel(input, rois):
    t = jnp.transpose(input, (0, 2, 3, 1)).reshape(N_IMG * H * W, C)
    blk = np.empty(32, dtype=np.int32)
    blk[0::2] = np.arange(16)
    blk[1::2] = np.arange(16, 32)
    perm = np.concatenate([blk + 32 * g for g in range(C // 32)])
    t = t[:, perm]
    tb = jnp.concatenate(
        [t, jnp.roll(t, -1, axis=0)], axis=1).astype(jnp.bfloat16)
    table = lax.bitcast_convert_type(
        tb.reshape(N_IMG * H * W, C, 2), jnp.int32)
    rois_p = jnp.pad(rois, ((0, RPAD - rois.shape[0]), (0, 0)))
    idx4, w4 = _coords(rois_p)
    wb = w4.astype(jnp.bfloat16)
    wpk = lax.bitcast_convert_type(jnp.stack([wb, wb], axis=-1), jnp.int32)
    out = _sc_gather_fn()(
        table,
        idx4.reshape(RPAD, 2, NSUB, SUB),
        wpk.reshape(RPAD, 4 * NSAMP),
    )
    return out[:R].reshape(R, C, OUT_H, OUT_W)


# speedup vs baseline: 1.0510x; 1.0510x over previous
import functools

import numpy as np
import jax
import jax.numpy as jnp
from jax import lax
from jax.experimental import pallas as pl
from jax.experimental.pallas import tpu as pltpu
from jax.experimental.pallas import tpu_sc as plsc

OUT_H, OUT_W = 8, 32
NSAMP = OUT_H * OUT_W
SPATIAL_SCALE = 0.25
N_IMG, C, H, W = 2, 128, 160, 160
R = 1000
RPAD = 1024
NC, NS = 2, 16
WORKERS = NC * NS
RPW = RPAD // WORKERS
SUB = 32
NSUB = NSAMP // SUB


def _coords_body(rois_ref, idx_ref, w_ref):
    r = rois_ref[...]
    b = r[:, 0:1].astype(jnp.int32)

    def pcol(i):
        return r[:, 1 + i:2 + i] * SPATIAL_SCALE

    px = [pcol(2 * j) for j in range(8)]
    py = [pcol(2 * j + 1) for j in range(8)]

    rb = r.shape[0]
    pos = lax.broadcasted_iota(jnp.int32, (rb, NSAMP), 1)
    u = (pos % OUT_W).astype(jnp.float32) * (1.0 / OUT_W)
    v = (pos // OUT_W).astype(jnp.float32) * (1.0 / OUT_H)

    def bez(p0, p1, p2, p3, t):
        omt = 1.0 - t
        return (omt * omt * omt) * p0 + 3.0 * (omt * omt) * t * p1 \
            + 3.0 * omt * (t * t) * p2 + (t * t * t) * p3

    x0 = bez(px[0], px[1], px[2], px[3], u)
    y0 = bez(py[0], py[1], py[2], py[3], u)
    x1 = bez(px[4], px[5], px[6], px[7], u)
    y1 = bez(py[4], py[5], py[6], py[7], u)
    xc = x1 * v + x0 * (1.0 - v) - 0.5
    yc = y1 * v + y0 * (1.0 - v) - 0.5

    roi_w = jnp.maximum(jnp.abs(px[0] - px[3]), jnp.abs(px[4] - px[7]))
    roi_h = jnp.maximum(jnp.abs(py[0] - py[4]), jnp.abs(py[3] - py[7]))
    bin_h = roi_h * (1.0 / OUT_H)
    bin_w = roi_w * (1.0 / OUT_W)
    ys = yc - 0.5 * bin_h + 0.5 * bin_h
    xs = xc - 0.5 * bin_w + 0.5 * bin_w

    valid = (ys >= -1.0) & (ys <= float(H)) & (xs >= -1.0) & (xs <= float(W))
    y = jnp.maximum(ys, 0.0)
    x = jnp.maximum(xs, 0.0)
    yl = jnp.minimum(y.astype(jnp.int32), H - 1)
    xl = jnp.minimum(x.astype(jnp.int32), W - 1)
    yh = jnp.minimum(yl + 1, H - 1)
    xh = jnp.minimum(xl + 1, W - 1)
    y = jnp.where(yl >= H - 1, yl.astype(jnp.float32), y)
    x = jnp.where(xl >= W - 1, xl.astype(jnp.float32), x)
    ly = y - yl.astype(jnp.float32)
    lx = x - xl.astype(jnp.float32)
    hy = 1.0 - ly
    hx = 1.0 - lx
    vf = valid.astype(jnp.float32)

    w_ref[:, 0, :] = hy * hx * vf
    w_ref[:, 1, :] = hy * lx * vf
    w_ref[:, 2, :] = ly * hx * vf
    w_ref[:, 3, :] = ly * lx * vf
    base = b * (H * W)
    idx_ref[:, 0, :] = base + yl * W + xl
    idx_ref[:, 1, :] = base + yh * W + xl


_RB = 128


def _coords(rois_p):
    return pl.pallas_call(
        _coords_body,
        grid=(RPAD // _RB,),
        in_specs=[pl.BlockSpec((_RB, 17), lambda i: (i, 0))],
        out_specs=[
            pl.BlockSpec((_RB, 2, NSAMP), lambda i: (i, 0, 0)),
            pl.BlockSpec((_RB, 4, NSAMP), lambda i: (i, 0, 0)),
        ],
        out_shape=[
            jax.ShapeDtypeStruct((RPAD, 2, NSAMP), jnp.int32),
            jax.ShapeDtypeStruct((RPAD, 4, NSAMP), jnp.float32),
        ],
    )(rois_p)


@functools.cache
def _sc_gather_fn():
    return functools.partial(
        pl.kernel,
        mesh=plsc.VectorSubcoreMesh(core_axis_name="c", subcore_axis_name="s"),
        out_type=jax.ShapeDtypeStruct((RPAD, C, NSAMP), jnp.float32),
        compiler_params=pltpu.CompilerParams(needs_layout_passes=False),
        scratch_types=[
            pltpu.VMEM((2, NSUB, SUB), jnp.int32),
            pltpu.VMEM((4 * NSAMP,), jnp.int32),
            pltpu.VMEM((2, SUB, C), jnp.int32),
            pltpu.VMEM((2, SUB, C), jnp.int32),
            pltpu.VMEM((C, NSAMP + 1), jnp.float32),
            pltpu.SemaphoreType.DMA,
            pltpu.SemaphoreType.DMA,
        ],
    )(_sc_gather_body)


def _sc_gather_body(table, idxs, ws, out, idx_v, w_v, stage0, stage1, acc,
                    sem0, sem1):
    wid = lax.axis_index("s") * NC + lax.axis_index("c")

    def issue(j, stage, sem):
        return [
            pltpu.async_copy(table.at[idx_v.at[k, j]], stage.at[k], sem)
            for k in range(2)
        ]

    def drain(stage, sem):
        for k in range(2):
            pltpu.make_async_copy(table.at[idx_v.at[k, 0]], stage.at[k],
                                  sem).wait()

    def compute(j, stage):
        wbase = j * SUB
        wvecs = [[w_v[pl.ds(k * NSAMP + wbase + h * 16, 16)] for h in range(2)]
                 for k in range(4)]
        ch_vecs = [lax.iota(jnp.int32, 16) + 32 * g for g in range(C // 32)]
        for s in range(SUB):
            wgt = [plsc.bitcast(
                jnp.full((16,), wvecs[k][s // 16][s % 16], jnp.int32),
                jnp.bfloat16) for k in range(4)]
            col_vec = jnp.full((16,), wbase + s, jnp.int32)
            for g in range(C // 32):
                accp = None
                for row, half, wi in ((0, 0, 0), (0, 1, 1), (1, 0, 2),
                                      (1, 1, 3)):
                    chunk = plsc.bitcast(
                        stage[row, s, pl.ds(half * (C // 2) + 16 * g, 16)],
                        jnp.bfloat16)
                    term = chunk * wgt[wi]
                    accp = term if accp is None else accp + term
                lo, hi = plsc.unpack(accp, format=plsc.PackFormat.INTERLEAVED)
                plsc.store_scatter(acc, [ch_vecs[g], col_vec], lo)
                plsc.store_scatter(acc, [ch_vecs[g] + 16, col_vec], hi)

    def roi_body(i, carry):
        roi = wid * RPW + i
        pltpu.sync_copy(idxs.at[roi], idx_v)
        pltpu.sync_copy(ws.at[roi], w_v)
        issue(0, stage0, sem0)

        def pair_body(jj, c2):
            j0 = jj * 2
            issue(j0 + 1, stage1, sem1)
            drain(stage0, sem0)
            compute(j0, stage0)

            @pl.when(jj < NSUB // 2 - 1)
            def _():
                issue(j0 + 2, stage0, sem0)

            drain(stage1, sem1)
            compute(j0 + 1, stage1)
            return c2

        lax.fori_loop(0, NSUB // 2, pair_body, 0)
        pltpu.sync_copy(acc.at[:, pl.ds(0, NSAMP)], out.at[roi])
        return carry

    lax.fori_loop(0, RPW, roi_body, 0)


def kernel(input, rois):
    t = jnp.transpose(input, (0, 2, 3, 1)).reshape(N_IMG * H * W, C)
    blk = np.empty(32, dtype=np.int32)
    blk[0::2] = np.arange(16)
    blk[1::2] = np.arange(16, 32)
    perm = np.concatenate([blk + 32 * g for g in range(C // 32)])
    t = t[:, perm]
    tb = jnp.concatenate(
        [t, jnp.roll(t, -1, axis=0)], axis=1).astype(jnp.bfloat16)
    table = lax.bitcast_convert_type(
        tb.reshape(N_IMG * H * W, C, 2), jnp.int32)
    rois_p = jnp.pad(rois, ((0, RPAD - rois.shape[0]), (0, 0)))
    idx4, w4 = _coords(rois_p)
    wb = w4.astype(jnp.bfloat16)
    wpk = lax.bitcast_convert_type(jnp.stack([wb, wb], axis=-1), jnp.int32)
    out = _sc_gather_fn()(
        table,
        idx4.reshape(RPAD, 2, NSUB, SUB),
        wpk.reshape(RPAD, 4 * NSAMP),
    )
    return out[:R].reshape(R, C, OUT_H, OUT_W)
